# baseline (speedup 1.0000x reference)
import functools

import jax
import jax.numpy as jnp
from jax import lax
from jax.experimental import pallas as pl
from jax.experimental.pallas import tpu as pltpu
from jax.experimental.pallas import tpu_sc as plsc

NN = 10000
NP = 10240
D = 128
HD = 64
LANES = 16
CHUNK = 128
NT = 16
C = 160
CH = C // 2
EPT = C * CHUNK
EP = EPT * NT
RPS = NP // NT
NRING = 6

_mesh = plsc.VectorSubcoreMesh(core_axis_name="c", subcore_axis_name="s")
_sc_params = pltpu.CompilerParams(needs_layout_passes=False,
                                  use_tc_tiling_on_sc=False)


def _bcast_entry(ref2d, r, e):
    ri = jnp.full((LANES,), r, dtype=jnp.int32)
    ei = jnp.full((LANES,), e, dtype=jnp.int32)
    return plsc.load_gather(ref2d, [ri, ei])


def _rsqrt16(x):
    y = plsc.bitcast(
        jnp.int32(0x5F3759DF) - (plsc.bitcast(x, jnp.int32) >> 1),
        jnp.float32,
    )
    hx = x * 0.5
    for _ in range(3):
        y = y * (1.5 - hx * y * y)
    return y


@functools.partial(
    pl.kernel,
    out_type=[
        jax.ShapeDtypeStruct((NT, C, CHUNK), jnp.float32),
        jax.ShapeDtypeStruct((2, NP), jnp.float32),
    ],
    mesh=_mesh,
    compiler_params=_sc_params,
    scratch_types=[
        pltpu.VMEM((C, CHUNK), jnp.int32),
        pltpu.VMEM((C, CHUNK), jnp.float32),
        pltpu.VMEM((CH, CHUNK), jnp.int32),
        pltpu.VMEM((CH, CHUNK), jnp.float32),
        pltpu.VMEM((NP,), jnp.float32),
        pltpu.VMEM((RPS,), jnp.float32),
        pltpu.VMEM_SHARED((NP,), jnp.float32),
    ],
)
def _prep_kernel(src_hbm, dst_hbm, w_hbm, norm_hbm, dinv_hbm,
                 dst_v, w_v, src_v, norm_v, dinv_v, buf_v, acc):
    cc = lax.axis_index("c")
    ss = lax.axis_index("s")
    half = pl.ds(cc * CH, CH)
    myrows = pl.ds(ss * RPS, RPS)
    for g in range(RPS // LANES):
        buf_v[pl.ds(g * LANES, LANES)] = jnp.zeros((LANES,), jnp.float32)
    pltpu.sync_copy(buf_v, acc.at[myrows])
    plsc.subcore_barrier()
    pltpu.sync_copy(dst_hbm.at[ss], dst_v)
    pltpu.sync_copy(w_hbm.at[ss], w_v)
    pltpu.sync_copy(src_hbm.at[ss, half], src_v)

    def body(c, carry):
        pltpu.sync_copy(w_v.at[c], acc.at[dst_v.at[c]], add=True)
        return carry

    lax.fori_loop(0, C, body, 0)
    plsc.subcore_barrier()
    pltpu.sync_copy(acc.at[myrows], buf_v)
    for g in range(RPS // LANES):
        sl = pl.ds(g * LANES, LANES)
        buf_v[sl] = _rsqrt16(buf_v[sl] + 1.0)
    pltpu.sync_copy(buf_v, acc.at[myrows])
    pltpu.sync_copy(buf_v, dinv_hbm.at[cc, myrows])
    plsc.subcore_barrier()
    pltpu.sync_copy(acc, dinv_v)

    @plsc.parallel_loop(0, CH, step=1, unroll=2)
    def _(c):
        for g in range(CHUNK // LANES):
            sl = pl.ds(g * LANES, LANES)
            si = src_v[c, sl]
            di = dst_v[cc * CH + c, sl]
            wv = w_v[cc * CH + c, sl]
            norm_v[c, sl] = (
                plsc.load_gather(dinv_v, [si])
                * wv
                * plsc.load_gather(dinv_v, [di])
            )

    pltpu.sync_copy(norm_v, norm_hbm.at[ss, half])


@functools.partial(
    pl.kernel,
    out_type=jax.ShapeDtypeStruct((2, NP, HD), jnp.float32),
    mesh=_mesh,
    compiler_params=_sc_params,
    scratch_types=[
        pltpu.VMEM((C, CHUNK), jnp.int32),
        pltpu.VMEM((C, CHUNK), jnp.int32),
        pltpu.VMEM((NRING, CHUNK), jnp.float32),
        pltpu.VMEM((CHUNK, HD), jnp.float32),
        pltpu.VMEM((CHUNK, HD), jnp.float32),
        pltpu.VMEM((CHUNK, HD), jnp.float32),
        pltpu.VMEM((CHUNK, HD), jnp.float32),
        pltpu.VMEM_SHARED((NP, HD), jnp.float32),
        pltpu.SemaphoreType.DMA,
        pltpu.SemaphoreType.DMA,
        pltpu.SemaphoreType.DMA,
        pltpu.SemaphoreType.DMA,
        pltpu.SemaphoreType.DMA,
    ],
)
def _msg_kernel(xw_hbm, src_hbm, dst_hbm, norm_hbm, out_hbm,
                src_v, dst_v, ring_v, g0, g1, s0, s1, acc,
                semg0, semg1, sems0, sems1, semn):
    cc = lax.axis_index("c")
    ss = lax.axis_index("s")
    xw_c = xw_hbm.at[cc]
    norm_t = norm_hbm.at[ss]

    def zbody(r, carry):
        for j in range(HD // LANES):
            s0[r, pl.ds(j * LANES, LANES)] = jnp.zeros((LANES,), jnp.float32)
        return carry

    lax.fori_loop(0, CHUNK, zbody, 0)
    for k in range(RPS // CHUNK):
        pltpu.sync_copy(s0, acc.at[pl.ds(ss * RPS + k * CHUNK, CHUNK)])
    plsc.subcore_barrier()

    pltpu.sync_copy(src_hbm.at[ss], src_v)
    pltpu.sync_copy(dst_hbm.at[ss], dst_v)
    pltpu.sync_copy(norm_t.at[pl.ds(0, NRING)], ring_v)

    def scale(c, gbuf, sbuf):
        slot = lax.rem(c, NRING)

        @plsc.parallel_loop(0, CHUNK, step=1, unroll=16)
        def _(e):
            bc = _bcast_entry(ring_v, slot, e)
            for j in range(HD // LANES):
                sl = pl.ds(j * LANES, LANES)
                sbuf[e, sl] = gbuf[e, sl] * bc

    def fetch_norm(c):
        nxt = lax.rem(c + NRING, C)
        pltpu.async_copy(norm_t.at[nxt], ring_v.at[lax.rem(c, NRING)], semn)

    bufs = ((g0, s0, semg0, sems0), (g1, s1, semg1, sems1))

    pltpu.async_copy(xw_c.at[src_v.at[0]], g0, semg0)
    pltpu.async_copy(xw_c.at[src_v.at[1]], g1, semg1)
    for p in range(2):
        gb, sb, sg, sc = bufs[p]
        pltpu.make_async_copy(xw_c.at[src_v.at[p]], gb, sg).wait()
        scale(p, gb, sb)
        fetch_norm(p)
        pltpu.async_copy(xw_c.at[src_v.at[p + 2]], gb, sg)
        pltpu.async_copy(sb, acc.at[dst_v.at[p]], sc, add=True)

    def body(i, carry):
        for p in range(2):
            c = 2 * i + p
            gb, sb, sg, sc = bufs[p]
            pltpu.make_async_copy(xw_c.at[src_v.at[c]], gb, sg).wait()
            pltpu.make_async_copy(sb, acc.at[dst_v.at[c - 2]], sc).wait()
            pltpu.make_async_copy(norm_t.at[c], ring_v.at[0], semn).wait()
            scale(c, gb, sb)
            fetch_norm(c)
            pltpu.async_copy(xw_c.at[src_v.at[lax.rem(c + 2, C)]], gb, sg)
            pltpu.async_copy(sb, acc.at[dst_v.at[c]], sc, add=True)
        return carry

    lax.fori_loop(1, C // 2, body, 0)
    pltpu.make_async_copy(xw_c.at[src_v.at[0]], g0, semg0).wait()
    pltpu.make_async_copy(xw_c.at[src_v.at[1]], g1, semg1).wait()
    pltpu.make_async_copy(s0, acc.at[dst_v.at[C - 2]], sems0).wait()
    pltpu.make_async_copy(s1, acc.at[dst_v.at[C - 1]], sems1).wait()
    for _ in range(2):
        pltpu.make_async_copy(norm_t.at[0], ring_v.at[0], semn).wait()
    plsc.subcore_barrier()
    for k in range(RPS // CHUNK):
        row = pl.ds(ss * RPS + k * CHUNK, CHUNK)
        pltpu.sync_copy(acc.at[row], g0)
        pltpu.sync_copy(g0, out_hbm.at[cc, row])


def _mm_body(x_ref, w_ref, o_ref):
    o_ref[0] = jnp.dot(x_ref[...], w_ref[0],
                       preferred_element_type=jnp.float32)


def _matmul_split(x, w_split, bm=512):
    m, k = x.shape
    nh = w_split.shape[2]
    return pl.pallas_call(
        _mm_body,
        grid=(m // bm, 2),
        in_specs=[
            pl.BlockSpec((bm, k), lambda i, j: (i, 0)),
            pl.BlockSpec((1, k, nh), lambda i, j: (j, 0, 0)),
        ],
        out_specs=pl.BlockSpec((1, bm, nh), lambda i, j: (j, i, 0)),
        out_shape=jax.ShapeDtypeStruct((2, m, nh), jnp.float32),
    )(x, w_split)


def _fuse_h(a0, a1, x0, x1, dinv, b):
    dv2 = dinv * dinv
    h0 = a0[0] + x0[0] * dv2
    h1 = a1[0] + x1[0] * dv2
    return jnp.concatenate([h0, h1], axis=1) + b


def _layer_body(a0_ref, a1_ref, x0_ref, x1_ref, dinv_ref, b_ref, w_ref,
                o_ref):
    h = _fuse_h(a0_ref[...], a1_ref[...], x0_ref[...], x1_ref[...],
                dinv_ref[...], b_ref[...])
    h = jnp.maximum(h, 0.0)
    o_ref[0] = jnp.dot(h, w_ref[0], preferred_element_type=jnp.float32)


def _layer_mm(acc, xw, dinv_col, b, w_split, bm=512):
    m = acc.shape[1]
    nh = w_split.shape[2]
    half = lambda p: pl.BlockSpec((1, bm, HD), lambda i, j, p=p: (p, i, 0))
    return pl.pallas_call(
        _layer_body,
        grid=(m // bm, 2),
        in_specs=[
            half(0),
            half(1),
            half(0),
            half(1),
            pl.BlockSpec((bm, 1), lambda i, j: (i, 0)),
            pl.BlockSpec((1, D), lambda i, j: (0, 0)),
            pl.BlockSpec((1, D, nh), lambda i, j: (j, 0, 0)),
        ],
        out_specs=pl.BlockSpec((1, bm, nh), lambda i, j: (j, i, 0)),
        out_shape=jax.ShapeDtypeStruct((2, m, nh), jnp.float32),
    )(acc, acc, xw, xw, dinv_col, b, w_split)


def _head_body(a0_ref, a1_ref, x0_ref, x1_ref, dinv_ref, b_ref,
               wmu_ref, bmu_ref, wlv_ref, blv_ref, mu_ref, lv_ref):
    h = _fuse_h(a0_ref[...], a1_ref[...], x0_ref[...], x1_ref[...],
                dinv_ref[...], b_ref[...])
    mu_ref[...] = (
        jnp.dot(h, wmu_ref[...], preferred_element_type=jnp.float32)
        + bmu_ref[...]
    )
    lv_ref[...] = (
        jnp.dot(h, wlv_ref[...], preferred_element_type=jnp.float32)
        + blv_ref[...]
    )


def _heads(acc, xw, dinv_col, b, wmu, bmu, wlv, blv, bm=512):
    m = acc.shape[1]
    n = wmu.shape[1]
    half = lambda p: pl.BlockSpec((1, bm, HD), lambda i, p=p: (p, i, 0))
    return pl.pallas_call(
        _head_body,
        grid=(m // bm,),
        in_specs=[
            half(0),
            half(1),
            half(0),
            half(1),
            pl.BlockSpec((bm, 1), lambda i: (i, 0)),
            pl.BlockSpec((1, D), lambda i: (0, 0)),
            pl.BlockSpec((D, n), lambda i: (0, 0)),
            pl.BlockSpec((1, n), lambda i: (0, 0)),
            pl.BlockSpec((D, n), lambda i: (0, 0)),
            pl.BlockSpec((1, n), lambda i: (0, 0)),
        ],
        out_specs=[
            pl.BlockSpec((bm, n), lambda i: (i, 0)),
            pl.BlockSpec((bm, n), lambda i: (i, 0)),
        ],
        out_shape=[
            jax.ShapeDtypeStruct((m, n), jnp.float32),
            jax.ShapeDtypeStruct((m, n), jnp.float32),
        ],
    )(acc, acc, xw, xw, dinv_col, b, wmu, bmu, wlv, blv)


def kernel(x, edge_index, weights, W1, b1, W2, b2, W_mu, b_mu, W_lv, b_lv):
    src = edge_index[0]
    dst = edge_index[1]
    e = src.shape[0]
    pad = EP - e
    srcp = jnp.concatenate([src, jnp.zeros((pad,), jnp.int32)])
    dstp = jnp.concatenate([dst, jnp.full((pad,), NP - 1, jnp.int32)])
    wp = jnp.concatenate([weights, jnp.zeros((pad,), jnp.float32)])
    src3 = srcp.reshape(NT, C, CHUNK)
    dst3 = dstp.reshape(NT, C, CHUNK)
    w3 = wp.reshape(NT, C, CHUNK)

    x_pad = jnp.pad(x, ((0, NP - x.shape[0]), (0, 0)))

    norm3, dinv2 = _prep_kernel(src3, dst3, w3)
    dinv_col = dinv2[0].reshape(NP, 1)

    W1s = W1.reshape(D, 2, HD).transpose(1, 0, 2)
    W2s = W2.reshape(D, 2, HD).transpose(1, 0, 2)
    xw1 = _matmul_split(x_pad, W1s)
    acc1 = _msg_kernel(xw1, src3, dst3, norm3)
    xw2 = _layer_mm(acc1, xw1, dinv_col, b1.reshape(1, D), W2s)
    acc2 = _msg_kernel(xw2, src3, dst3, norm3)
    mu_p, lv_p = _heads(acc2, xw2, dinv_col, b2.reshape(1, D),
                        W_mu, b_mu.reshape(1, -1), W_lv, b_lv.reshape(1, -1))
    n = x.shape[0]
    return (mu_p[:n], lv_p[:n])


# speedup vs baseline: 1.5219x; 1.5219x over previous
import functools

import jax
import jax.numpy as jnp
from jax import lax
from jax.experimental import pallas as pl
from jax.experimental.pallas import tpu as pltpu
from jax.experimental.pallas import tpu_sc as plsc

NN = 10000
NP = 10240
D = 128
HD = 64
LANES = 16
CHUNK = 128
NT = 16
C = 160
CH = C // 2
EPT = C * CHUNK
EP = EPT * NT
RPS = NP // NT
NRING = 6

_mesh = plsc.VectorSubcoreMesh(core_axis_name="c", subcore_axis_name="s")
_sc_params = pltpu.CompilerParams(needs_layout_passes=False,
                                  use_tc_tiling_on_sc=False)


def _bcast_entry(ref2d, r, e):
    ri = jnp.full((LANES,), r, dtype=jnp.int32)
    ei = jnp.full((LANES,), e, dtype=jnp.int32)
    return plsc.load_gather(ref2d, [ri, ei])


def _rsqrt16(x):
    y = plsc.bitcast(
        jnp.int32(0x5F3759DF) - (plsc.bitcast(x, jnp.int32) >> 1),
        jnp.float32,
    )
    hx = x * 0.5
    for _ in range(3):
        y = y * (1.5 - hx * y * y)
    return y


@functools.partial(
    pl.kernel,
    out_type=[
        jax.ShapeDtypeStruct((NT, C, 3, CHUNK), jnp.int32),
        jax.ShapeDtypeStruct((2, NP), jnp.float32),
    ],
    mesh=_mesh,
    compiler_params=_sc_params,
    scratch_types=[
        pltpu.VMEM((C, CHUNK), jnp.int32),
        pltpu.VMEM((C, CHUNK), jnp.float32),
        pltpu.VMEM((CH, CHUNK), jnp.int32),
        pltpu.VMEM((CH, 3, CHUNK), jnp.int32),
        pltpu.VMEM((NP,), jnp.float32),
        pltpu.VMEM((RPS,), jnp.float32),
        pltpu.VMEM_SHARED((NP,), jnp.float32),
        pltpu.SemaphoreType.DMA,
    ],
)
def _prep_kernel(src_hbm, dst_hbm, w_hbm, meta_hbm, dinv_hbm,
                 dst_v, w_v, src_v, mv, dinv_v, buf_v, acc, semd):
    cc = lax.axis_index("c")
    ss = lax.axis_index("s")
    half = pl.ds(cc * CH, CH)
    myrows = pl.ds(ss * RPS, RPS)
    for g in range(RPS // LANES):
        buf_v[pl.ds(g * LANES, LANES)] = jnp.zeros((LANES,), jnp.float32)
    pltpu.sync_copy(buf_v, acc.at[myrows])
    plsc.subcore_barrier()
    pltpu.sync_copy(dst_hbm.at[ss], dst_v)
    pltpu.sync_copy(w_hbm.at[ss], w_v)
    pltpu.sync_copy(src_hbm.at[ss, half], src_v)

    def body(c, carry):
        pltpu.async_copy(w_v.at[c], acc.at[dst_v.at[c]], semd, add=True)
        return carry

    lax.fori_loop(0, C, body, 0)

    def drain(c, carry):
        pltpu.make_async_copy(w_v.at[0], acc.at[dst_v.at[0]], semd).wait()
        return carry

    lax.fori_loop(0, C, drain, 0)
    plsc.subcore_barrier()
    pltpu.sync_copy(acc.at[myrows], buf_v)
    for g in range(RPS // LANES):
        sl = pl.ds(g * LANES, LANES)
        buf_v[sl] = _rsqrt16(buf_v[sl] + 1.0)
    pltpu.sync_copy(buf_v, acc.at[myrows])
    pltpu.sync_copy(buf_v, dinv_hbm.at[cc, myrows])
    plsc.subcore_barrier()
    pltpu.sync_copy(acc, dinv_v)

    @plsc.parallel_loop(0, CH, step=1, unroll=2)
    def _(c):
        for g in range(CHUNK // LANES):
            sl = pl.ds(g * LANES, LANES)
            si = src_v[c, sl]
            di = dst_v[cc * CH + c, sl]
            wv = w_v[cc * CH + c, sl]
            nv = (
                plsc.load_gather(dinv_v, [si])
                * wv
                * plsc.load_gather(dinv_v, [di])
            )
            mv[c, 0, sl] = si
            mv[c, 1, sl] = di
            mv[c, 2, sl] = plsc.bitcast(nv, jnp.int32)

    pltpu.sync_copy(mv, meta_hbm.at[ss, half])


@functools.partial(
    pl.kernel,
    out_type=jax.ShapeDtypeStruct((2, NP, HD), jnp.float32),
    mesh=_mesh,
    compiler_params=_sc_params,
    scratch_types=[
        pltpu.VMEM((NRING, 3, CHUNK), jnp.int32),
        pltpu.VMEM((CHUNK, HD), jnp.float32),
        pltpu.VMEM((CHUNK, HD), jnp.float32),
        pltpu.VMEM((CHUNK, HD), jnp.float32),
        pltpu.VMEM((CHUNK, HD), jnp.float32),
        pltpu.VMEM_SHARED((NP, HD), jnp.float32),
        pltpu.VMEM_SHARED((NP, HD), jnp.float32),
        pltpu.SemaphoreType.DMA,
        pltpu.SemaphoreType.DMA,
        pltpu.SemaphoreType.DMA,
        pltpu.SemaphoreType.DMA,
        pltpu.SemaphoreType.DMA,
    ],
)
def _msg_kernel(xw_hbm, meta_hbm, out_hbm,
                ring_v, g0, g1, s0, s1, xw_sp, acc,
                semg0, semg1, sems0, sems1, semm):
    cc = lax.axis_index("c")
    ss = lax.axis_index("s")
    meta_t = meta_hbm.at[ss]
    myrows = pl.ds(ss * RPS, RPS)

    def zbody(r, carry):
        for j in range(HD // LANES):
            s0[r, pl.ds(j * LANES, LANES)] = jnp.zeros((LANES,), jnp.float32)
        return carry

    lax.fori_loop(0, CHUNK, zbody, 0)
    for k in range(RPS // CHUNK):
        pltpu.sync_copy(s0, acc.at[pl.ds(ss * RPS + k * CHUNK, CHUNK)])
    pltpu.sync_copy(xw_hbm.at[cc, myrows], xw_sp.at[myrows])
    plsc.subcore_barrier()

    pltpu.sync_copy(meta_t.at[pl.ds(0, NRING)], ring_v)

    def scale(c, gbuf, sbuf):
        slot = lax.rem(c, NRING)

        @plsc.parallel_loop(0, CHUNK, step=1, unroll=16)
        def _(e):
            bc = plsc.bitcast(
                plsc.load_gather(
                    ring_v,
                    [jnp.full((LANES,), slot, jnp.int32),
                     jnp.full((LANES,), 2, jnp.int32),
                     jnp.full((LANES,), e, jnp.int32)],
                ),
                jnp.float32,
            )
            for j in range(HD // LANES):
                sl = pl.ds(j * LANES, LANES)
                sbuf[e, sl] = gbuf[e, sl] * bc

    def fetch_meta(c):
        nxt = lax.rem(c + NRING, C)
        pltpu.async_copy(meta_t.at[nxt], ring_v.at[lax.rem(c, NRING)], semm)

    def src_at(c):
        return ring_v.at[lax.rem(c, NRING), 0]

    def dst_at(c):
        return ring_v.at[lax.rem(c, NRING), 1]

    bufs = ((g0, s0, semg0, sems0), (g1, s1, semg1, sems1))

    pltpu.async_copy(xw_sp.at[src_at(0)], g0, semg0)
    pltpu.async_copy(xw_sp.at[src_at(1)], g1, semg1)
    for p in range(2):
        gb, sb, sg, sc = bufs[p]
        pltpu.make_async_copy(xw_sp.at[src_at(p)], gb, sg).wait()
        scale(p, gb, sb)
        pltpu.async_copy(xw_sp.at[src_at(p + 2)], gb, sg)
        pltpu.async_copy(sb, acc.at[dst_at(p)], sc, add=True)
        fetch_meta(p)

    def body(i, carry):
        for p in range(2):
            c = 2 * i + p
            gb, sb, sg, sc = bufs[p]
            pltpu.make_async_copy(xw_sp.at[src_at(c)], gb, sg).wait()
            pltpu.make_async_copy(sb, acc.at[dst_at(c)], sc).wait()
            pltpu.make_async_copy(meta_t.at[c], ring_v.at[0], semm).wait()
            scale(c, gb, sb)
            pltpu.async_copy(xw_sp.at[src_at(lax.rem(c + 2, C))], gb, sg)
            pltpu.async_copy(sb, acc.at[dst_at(c)], sc, add=True)
            fetch_meta(c)
        return carry

    lax.fori_loop(1, C // 2, body, 0)
    pltpu.make_async_copy(xw_sp.at[src_at(0)], g0, semg0).wait()
    pltpu.make_async_copy(xw_sp.at[src_at(1)], g1, semg1).wait()
    pltpu.make_async_copy(s0, acc.at[dst_at(C - 2)], sems0).wait()
    pltpu.make_async_copy(s1, acc.at[dst_at(C - 1)], sems1).wait()
    for _ in range(2):
        pltpu.make_async_copy(meta_t.at[0], ring_v.at[0], semm).wait()
    plsc.subcore_barrier()
    for k in range(RPS // CHUNK):
        row = pl.ds(ss * RPS + k * CHUNK, CHUNK)
        pltpu.sync_copy(acc.at[row], g0)
        pltpu.sync_copy(g0, out_hbm.at[cc, row])


def _mm_body(x_ref, w_ref, o_ref):
    o_ref[0] = jnp.dot(x_ref[...], w_ref[0],
                       preferred_element_type=jnp.float32)


def _matmul_split(x, w_split, bm=512):
    m, k = x.shape
    nh = w_split.shape[2]
    return pl.pallas_call(
        _mm_body,
        grid=(m // bm, 2),
        in_specs=[
            pl.BlockSpec((bm, k), lambda i, j: (i, 0)),
            pl.BlockSpec((1, k, nh), lambda i, j: (j, 0, 0)),
        ],
        out_specs=pl.BlockSpec((1, bm, nh), lambda i, j: (j, i, 0)),
        out_shape=jax.ShapeDtypeStruct((2, m, nh), jnp.float32),
    )(x, w_split)


def _fuse_h(a0, a1, x0, x1, dinv, b):
    dv2 = dinv * dinv
    h0 = a0[0] + x0[0] * dv2
    h1 = a1[0] + x1[0] * dv2
    return jnp.concatenate([h0, h1], axis=1) + b


def _layer_body(a0_ref, a1_ref, x0_ref, x1_ref, dinv_ref, b_ref, w_ref,
                o_ref):
    h = _fuse_h(a0_ref[...], a1_ref[...], x0_ref[...], x1_ref[...],
                dinv_ref[...], b_ref[...])
    h = jnp.maximum(h, 0.0)
    o_ref[0] = jnp.dot(h, w_ref[0], preferred_element_type=jnp.float32)


def _layer_mm(acc, xw, dinv_col, b, w_split, bm=512):
    m = acc.shape[1]
    nh = w_split.shape[2]
    half = lambda p: pl.BlockSpec((1, bm, HD), lambda i, j, p=p: (p, i, 0))
    return pl.pallas_call(
        _layer_body,
        grid=(m // bm, 2),
        in_specs=[
            half(0),
            half(1),
            half(0),
            half(1),
            pl.BlockSpec((bm, 1), lambda i, j: (i, 0)),
            pl.BlockSpec((1, D), lambda i, j: (0, 0)),
            pl.BlockSpec((1, D, nh), lambda i, j: (j, 0, 0)),
        ],
        out_specs=pl.BlockSpec((1, bm, nh), lambda i, j: (j, i, 0)),
        out_shape=jax.ShapeDtypeStruct((2, m, nh), jnp.float32),
    )(acc, acc, xw, xw, dinv_col, b, w_split)


def _head_body(a0_ref, a1_ref, x0_ref, x1_ref, dinv_ref, b_ref,
               wmu_ref, bmu_ref, wlv_ref, blv_ref, mu_ref, lv_ref):
    h = _fuse_h(a0_ref[...], a1_ref[...], x0_ref[...], x1_ref[...],
                dinv_ref[...], b_ref[...])
    mu_ref[...] = (
        jnp.dot(h, wmu_ref[...], preferred_element_type=jnp.float32)
        + bmu_ref[...]
    )
    lv_ref[...] = (
        jnp.dot(h, wlv_ref[...], preferred_element_type=jnp.float32)
        + blv_ref[...]
    )


def _heads(acc, xw, dinv_col, b, wmu, bmu, wlv, blv, bm=512):
    m = acc.shape[1]
    n = wmu.shape[1]
    half = lambda p: pl.BlockSpec((1, bm, HD), lambda i, p=p: (p, i, 0))
    return pl.pallas_call(
        _head_body,
        grid=(m // bm,),
        in_specs=[
            half(0),
            half(1),
            half(0),
            half(1),
            pl.BlockSpec((bm, 1), lambda i: (i, 0)),
            pl.BlockSpec((1, D), lambda i: (0, 0)),
            pl.BlockSpec((D, n), lambda i: (0, 0)),
            pl.BlockSpec((1, n), lambda i: (0, 0)),
            pl.BlockSpec((D, n), lambda i: (0, 0)),
            pl.BlockSpec((1, n), lambda i: (0, 0)),
        ],
        out_specs=[
            pl.BlockSpec((bm, n), lambda i: (i, 0)),
            pl.BlockSpec((bm, n), lambda i: (i, 0)),
        ],
        out_shape=[
            jax.ShapeDtypeStruct((m, n), jnp.float32),
            jax.ShapeDtypeStruct((m, n), jnp.float32),
        ],
    )(acc, acc, xw, xw, dinv_col, b, wmu, bmu, wlv, blv)


def kernel(x, edge_index, weights, W1, b1, W2, b2, W_mu, b_mu, W_lv, b_lv):
    src = edge_index[0]
    dst = edge_index[1]
    e = src.shape[0]
    pad = EP - e
    srcp = jnp.concatenate([src, jnp.zeros((pad,), jnp.int32)])
    dstp = jnp.concatenate([dst, jnp.full((pad,), NP - 1, jnp.int32)])
    wp = jnp.concatenate([weights, jnp.zeros((pad,), jnp.float32)])
    src3 = srcp.reshape(NT, C, CHUNK)
    dst3 = dstp.reshape(NT, C, CHUNK)
    w3 = wp.reshape(NT, C, CHUNK)

    x_pad = jnp.pad(x, ((0, NP - x.shape[0]), (0, 0)))

    meta, dinv2 = _prep_kernel(src3, dst3, w3)
    dinv_col = dinv2[0].reshape(NP, 1)

    W1s = W1.reshape(D, 2, HD).transpose(1, 0, 2)
    W2s = W2.reshape(D, 2, HD).transpose(1, 0, 2)
    xw1 = _matmul_split(x_pad, W1s)
    acc1 = _msg_kernel(xw1, meta)
    xw2 = _layer_mm(acc1, xw1, dinv_col, b1.reshape(1, D), W2s)
    acc2 = _msg_kernel(xw2, meta)
    mu_p, lv_p = _heads(acc2, xw2, dinv_col, b2.reshape(1, D),
                        W_mu, b_mu.reshape(1, -1), W_lv, b_lv.reshape(1, -1))
    n = x.shape[0]
    return (mu_p[:n], lv_p[:n])
